# baseline (speedup 1.0000x reference)
import functools

import jax
import jax.numpy as jnp
from jax import lax
from jax.experimental import pallas as pl
from jax.experimental.pallas import tpu as pltpu
from jax.experimental.pallas import tpu_sc as plsc

N = 10000
E = 160000
D = 256
NP = 10240
E_PAD = 163840
EPW = E_PAD // 32
NBLK = EPW // 128
TBLK = E_PAD // 128
HALF = N // 2
SLAB = 5120
FB = 16
SB = 32
GB = 8
NCH = SB // GB
NSLOT = 2

_mesh = plsc.VectorSubcoreMesh(core_axis_name="c", subcore_axis_name="s",
                               num_cores=2, num_subcores=16)
_f32 = jnp.float32
_i32 = jnp.int32


def _rsqrt16(v):
    i = plsc.bitcast(v, _i32)
    i = jnp.full((16,), 0x5F3759DF, _i32) - lax.shift_right_arithmetic(
        i, jnp.full((16,), 1, _i32))
    y = plsc.bitcast(i, _f32)
    for _ in range(3):
        y = y * (1.5 - 0.5 * v * y * y)
    return y


@functools.partial(
    pl.kernel,
    out_type=[
        jax.ShapeDtypeStruct((2, NP), _f32),
        jax.ShapeDtypeStruct((2, NP), _f32),
    ],
    mesh=_mesh,
    compiler_params=pltpu.CompilerParams(needs_layout_passes=False),
    scratch_types=[
        pltpu.VMEM((128,), _i32),
        pltpu.VMEM((128,), _i32),
        pltpu.VMEM((128,), _f32),
        pltpu.VMEM((128,), _f32),
        pltpu.VMEM((NP // 16,), _f32),
        pltpu.VMEM_SHARED((NP,), _f32),
        pltpu.VMEM_SHARED((NP,), _f32),
    ],
)
def _stats(src_h, dst_h, attr_h, ssp_o, sdp_o,
           sb, db, ab, pb, zv, ssrc_sh, sdst_sh):
    cid = lax.axis_index("c")
    sid = lax.axis_index("s")
    w = cid * 16 + sid
    stripe = NP // 16

    zeros16 = jnp.zeros((16,), _f32)

    @pl.loop(0, stripe // 16)
    def _(i):
        zv[pl.ds(i * 16, 16)] = zeros16

    pltpu.sync_copy(zv, ssrc_sh.at[pl.ds(sid * stripe, stripe)])
    pltpu.sync_copy(zv, sdst_sh.at[pl.ds(sid * stripe, stripe)])
    plsc.subcore_barrier()

    @pl.loop(0, NBLK)
    def _(t):
        base = w * EPW + t * 128
        pltpu.sync_copy(src_h.at[pl.ds(base, 128)], sb)
        pltpu.sync_copy(dst_h.at[pl.ds(base, 128)], db)
        pltpu.sync_copy(attr_h.at[pl.ds(base, 128)], ab)

        @pl.loop(0, 8)
        def _(u):
            sl = pl.ds(u * 16, 16)
            pb[sl] = jnp.exp(ab[sl])

        pltpu.sync_copy(pb, ssrc_sh.at[sb], add=True)
        pltpu.sync_copy(pb, sdst_sh.at[db], add=True)

    plsc.subcore_barrier()
    pltpu.sync_copy(ssrc_sh.at[pl.ds(sid * stripe, stripe)],
                    ssp_o.at[cid, pl.ds(sid * stripe, stripe)])
    pltpu.sync_copy(sdst_sh.at[pl.ds(sid * stripe, stripe)],
                    sdp_o.at[cid, pl.ds(sid * stripe, stripe)])


@functools.partial(
    pl.kernel,
    out_type=jax.ShapeDtypeStruct((E_PAD,), _f32),
    mesh=_mesh,
    compiler_params=pltpu.CompilerParams(needs_layout_passes=False),
    scratch_types=[
        pltpu.VMEM((NP,), _f32),
        pltpu.VMEM((NP,), _f32),
        pltpu.VMEM((NP,), _f32),
        pltpu.VMEM((128,), _i32),
        pltpu.VMEM((128,), _i32),
        pltpu.VMEM((128,), _f32),
        pltpu.VMEM((128,), _f32),
    ],
)
def _coeffs(src_h, dst_h, attr_h, ssp_h, sdp_h, c_o,
            ss, sd, tmp, sb, db, ab, cb):
    cid = lax.axis_index("c")
    sid = lax.axis_index("s")
    w = cid * 16 + sid

    pltpu.sync_copy(ssp_h.at[0], ss)
    pltpu.sync_copy(ssp_h.at[1], tmp)

    @pl.loop(0, NP // 16)
    def _(i):
        sl = pl.ds(i * 16, 16)
        ss[sl] = ss[sl] + tmp[sl]

    pltpu.sync_copy(sdp_h.at[0], sd)
    pltpu.sync_copy(sdp_h.at[1], tmp)

    @pl.loop(0, NP // 16)
    def _(i):
        sl = pl.ds(i * 16, 16)
        sd[sl] = sd[sl] + tmp[sl]

    @pl.loop(0, NBLK)
    def _(t):
        base = w * EPW + t * 128
        pltpu.sync_copy(src_h.at[pl.ds(base, 128)], sb)
        pltpu.sync_copy(dst_h.at[pl.ds(base, 128)], db)
        pltpu.sync_copy(attr_h.at[pl.ds(base, 128)], ab)
        for u in range(8):
            sl = pl.ds(u * 16, 16)
            sv = plsc.load_gather(ss, [sb[sl]])
            dv = plsc.load_gather(sd, [db[sl]])
            v = sv * dv
            c = jnp.exp(ab[sl]) * _rsqrt16(v)
            cb[sl] = jnp.where(v > 0.0, c, 0.0)
        pltpu.sync_copy(cb, c_o.at[pl.ds(base, 128)])


@functools.partial(
    pl.kernel,
    out_type=jax.ShapeDtypeStruct((2, 16, SLAB * FB), _f32),
    mesh=_mesh,
    compiler_params=pltpu.CompilerParams(needs_layout_passes=False,
                                         use_tc_tiling_on_sc=False),
    scratch_types=[
        pltpu.VMEM((SLAB * FB,), _f32),
        pltpu.VMEM((NSLOT, GB * 128, FB), _f32),
        pltpu.VMEM((SB * 128,), _i32),
        pltpu.VMEM((SB * 128,), _i32),
        pltpu.VMEM((SB * 128,), _f32),
        pltpu.SemaphoreType.DMA,
        pltpu.SemaphoreType.DMA,
    ],
)
def _accum(x_h, src_h, dst_h, c_h, out_h,
           stripe, g, tsb, db, cb, sem0, sem1):
    cid = lax.axis_index("c")
    sid = lax.axis_index("s")
    lo = cid * HALF
    sems = (sem0, sem1)
    zeros16 = jnp.zeros((16,), _f32)
    iota16 = lax.iota(_i32, 16)
    sidv = jnp.broadcast_to(sid, (16,))

    @pl.loop(0, SLAB)
    def _(i):
        stripe[pl.ds(i * 16, 16)] = zeros16

    def fire(q, p):
        pltpu.async_copy(
            x_h.at[tsb.at[pl.ds(q * (GB * 128), GB * 128)]], g.at[p], sems[p])

    def drain(q, p):
        pltpu.make_async_copy(
            x_h.at[tsb.at[pl.ds(q * (GB * 128), GB * 128)]], g.at[p],
            sems[p]).wait()

    def finish(q, p):
        @pl.loop(0, GB * 8)
        def _(u):
            off = q * (GB * 128) + u * 16
            dvec = db[pl.ds(off, 16)]
            cvec = cb[pl.ds(off, 16)]
            m = (dvec >= lo) & (dvec < lo + HALF)
            ldv = jnp.where(m, dvec - lo, HALF) * 16
            cm = jnp.where(m, cvec, 0.0)
            for j in range(16):
                idx = jnp.broadcast_to(ldv[j], (16,)) + iota16
                val = g[p, u * 16 + j, pl.ds(0, 16)] * jnp.broadcast_to(
                    cm[j], (16,))
                plsc.addupdate_scatter(stripe, [idx], val)

    @pl.loop(0, TBLK // SB)
    def _(s):
        base = s * (SB * 128)
        pltpu.sync_copy(src_h.at[pl.ds(base, SB * 128)], tsb)
        pltpu.sync_copy(dst_h.at[pl.ds(base, SB * 128)], db)
        pltpu.sync_copy(c_h.at[pl.ds(base, SB * 128)], cb)

        @pl.loop(0, SB * 8)
        def _(v):
            sl = pl.ds(v * 16, 16)
            tsb[sl] = tsb[sl] * 16 + sidv

        for q in range(NSLOT - 1):
            fire(q, q)

        @pl.loop(0, NCH, step=NSLOT)
        def _(t):
            for b in range(NSLOT):
                tt = t + b

                @pl.when(tt + NSLOT - 1 < NCH)
                def _():
                    fire(tt + NSLOT - 1, (b + NSLOT - 1) % NSLOT)

                drain(tt, b)
                finish(tt, b)

    pltpu.sync_copy(stripe, out_h.at[cid, sid])


def kernel(x, edge_index, edge_attrs):
    src = edge_index[0]
    dst = edge_index[1]
    pad = E_PAD - E
    srcp = jnp.concatenate([src, jnp.zeros((pad,), _i32)])
    dstp = jnp.concatenate([dst, jnp.full((pad,), N, _i32)])
    attrp = jnp.concatenate([edge_attrs, jnp.full((pad,), -1e30, _f32)])
    ssp, sdp = _stats(srcp, dstp, attrp)
    c = _coeffs(srcp, dstp, attrp, ssp, sdp)
    x_rs = x.reshape(N * 16, FB)
    outp = _accum(x_rs, srcp, dstp, c)
    o = outp.reshape(2, 16, SLAB, FB).transpose(0, 2, 1, 3).reshape(
        2 * SLAB, D)
    return jnp.concatenate([o[:HALF], o[SLAB:SLAB + HALF]], axis=0)


# speedup vs baseline: 1.6009x; 1.6009x over previous
import functools

import jax
import jax.numpy as jnp
from jax import lax
from jax.experimental import pallas as pl
from jax.experimental.pallas import tpu as pltpu
from jax.experimental.pallas import tpu_sc as plsc

N = 10000
E = 160000
D = 256
NP = 10240
E_PAD = 163840
EPW = E_PAD // 32
NBLK = EPW // 128
TBLK = E_PAD // 128
HALF = N // 2
SLAB = 5120
FB = 16
SB = 64
CH = 512
NSLOT = 2

_mesh = plsc.VectorSubcoreMesh(core_axis_name="c", subcore_axis_name="s",
                               num_cores=2, num_subcores=16)
_f32 = jnp.float32
_i32 = jnp.int32


def _rsqrt16(v):
    i = plsc.bitcast(v, _i32)
    i = jnp.full((16,), 0x5F3759DF, _i32) - lax.shift_right_arithmetic(
        i, jnp.full((16,), 1, _i32))
    y = plsc.bitcast(i, _f32)
    for _ in range(3):
        y = y * (1.5 - 0.5 * v * y * y)
    return y


@functools.partial(
    pl.kernel,
    out_type=[
        jax.ShapeDtypeStruct((2, NP), _f32),
        jax.ShapeDtypeStruct((2, NP), _f32),
    ],
    mesh=_mesh,
    compiler_params=pltpu.CompilerParams(needs_layout_passes=False),
    scratch_types=[
        pltpu.VMEM((128,), _i32),
        pltpu.VMEM((128,), _i32),
        pltpu.VMEM((128,), _f32),
        pltpu.VMEM((128,), _f32),
        pltpu.VMEM((NP // 16,), _f32),
        pltpu.VMEM_SHARED((NP,), _f32),
        pltpu.VMEM_SHARED((NP,), _f32),
    ],
)
def _stats(src_h, dst_h, attr_h, ssp_o, sdp_o,
           sb, db, ab, pb, zv, ssrc_sh, sdst_sh):
    cid = lax.axis_index("c")
    sid = lax.axis_index("s")
    w = cid * 16 + sid
    stripe = NP // 16

    zeros16 = jnp.zeros((16,), _f32)

    @pl.loop(0, stripe // 16)
    def _(i):
        zv[pl.ds(i * 16, 16)] = zeros16

    pltpu.sync_copy(zv, ssrc_sh.at[pl.ds(sid * stripe, stripe)])
    pltpu.sync_copy(zv, sdst_sh.at[pl.ds(sid * stripe, stripe)])
    plsc.subcore_barrier()

    @pl.loop(0, NBLK)
    def _(t):
        base = w * EPW + t * 128
        pltpu.sync_copy(src_h.at[pl.ds(base, 128)], sb)
        pltpu.sync_copy(dst_h.at[pl.ds(base, 128)], db)
        pltpu.sync_copy(attr_h.at[pl.ds(base, 128)], ab)

        @pl.loop(0, 8)
        def _(u):
            sl = pl.ds(u * 16, 16)
            pb[sl] = jnp.exp(ab[sl])

        pltpu.sync_copy(pb, ssrc_sh.at[sb], add=True)
        pltpu.sync_copy(pb, sdst_sh.at[db], add=True)

    plsc.subcore_barrier()
    pltpu.sync_copy(ssrc_sh.at[pl.ds(sid * stripe, stripe)],
                    ssp_o.at[cid, pl.ds(sid * stripe, stripe)])
    pltpu.sync_copy(sdst_sh.at[pl.ds(sid * stripe, stripe)],
                    sdp_o.at[cid, pl.ds(sid * stripe, stripe)])


@functools.partial(
    pl.kernel,
    out_type=jax.ShapeDtypeStruct((E_PAD,), _f32),
    mesh=_mesh,
    compiler_params=pltpu.CompilerParams(needs_layout_passes=False),
    scratch_types=[
        pltpu.VMEM((NP,), _f32),
        pltpu.VMEM((NP,), _f32),
        pltpu.VMEM((NP,), _f32),
        pltpu.VMEM((128,), _i32),
        pltpu.VMEM((128,), _i32),
        pltpu.VMEM((128,), _f32),
        pltpu.VMEM((128,), _f32),
    ],
)
def _coeffs(src_h, dst_h, attr_h, ssp_h, sdp_h, c_o,
            ss, sd, tmp, sb, db, ab, cb):
    cid = lax.axis_index("c")
    sid = lax.axis_index("s")
    w = cid * 16 + sid

    pltpu.sync_copy(ssp_h.at[0], ss)
    pltpu.sync_copy(ssp_h.at[1], tmp)

    @pl.loop(0, NP // 16)
    def _(i):
        sl = pl.ds(i * 16, 16)
        ss[sl] = ss[sl] + tmp[sl]

    pltpu.sync_copy(sdp_h.at[0], sd)
    pltpu.sync_copy(sdp_h.at[1], tmp)

    @pl.loop(0, NP // 16)
    def _(i):
        sl = pl.ds(i * 16, 16)
        sd[sl] = sd[sl] + tmp[sl]

    @pl.loop(0, NBLK)
    def _(t):
        base = w * EPW + t * 128
        pltpu.sync_copy(src_h.at[pl.ds(base, 128)], sb)
        pltpu.sync_copy(dst_h.at[pl.ds(base, 128)], db)
        pltpu.sync_copy(attr_h.at[pl.ds(base, 128)], ab)
        for u in range(8):
            sl = pl.ds(u * 16, 16)
            sv = plsc.load_gather(ss, [sb[sl]])
            dv = plsc.load_gather(sd, [db[sl]])
            v = sv * dv
            c = jnp.exp(ab[sl]) * _rsqrt16(v)
            cb[sl] = jnp.where(v > 0.0, c, 0.0)
        pltpu.sync_copy(cb, c_o.at[pl.ds(base, 128)])


@functools.partial(
    pl.kernel,
    out_type=jax.ShapeDtypeStruct((2, 16, SLAB * FB), _f32),
    mesh=_mesh,
    compiler_params=pltpu.CompilerParams(needs_layout_passes=False,
                                         use_tc_tiling_on_sc=False),
    scratch_types=[
        pltpu.VMEM((SLAB * FB,), _f32),
        pltpu.VMEM((NSLOT, CH, FB), _f32),
        pltpu.VMEM((SB * 128 + 544,), _i32),
        pltpu.VMEM((SB * 128 + 544,), _i32),
        pltpu.VMEM((SB * 128 + 544,), _f32),
        pltpu.SemaphoreType.DMA,
        pltpu.SemaphoreType.DMA,
    ],
)
def _accum(x_h, src_h, dst_h, c_h, out_h,
           stripe, g, tsb, db, cb, sem0, sem1):
    cid = lax.axis_index("c")
    sid = lax.axis_index("s")
    lo = cid * HALF
    sems = (sem0, sem1)
    zeros16 = jnp.zeros((16,), _f32)
    iota16 = lax.iota(_i32, 16)
    sidv = jnp.broadcast_to(sid, (16,))

    @pl.loop(0, SLAB)
    def _(i):
        stripe[pl.ds(i * 16, 16)] = zeros16

    def fire(q, p):
        pltpu.async_copy(
            x_h.at[tsb.at[pl.ds(q * CH, CH)]], g.at[p], sems[p])

    def drain(q, p):
        pltpu.make_async_copy(
            x_h.at[tsb.at[pl.ds(q * CH, CH)]], g.at[p], sems[p]).wait()

    def finish(q, p):
        @pl.loop(0, CH // 16)
        def _(u):
            off = q * CH + u * 16
            ldv = db[pl.ds(off, 16)]
            cm = cb[pl.ds(off, 16)]
            for j in range(16):
                idx = jnp.broadcast_to(ldv[j], (16,)) + iota16
                val = g[p, u * 16 + j, pl.ds(0, 16)] * jnp.broadcast_to(
                    cm[j], (16,))
                plsc.addupdate_scatter(stripe, [idx], val)

    @pl.loop(0, TBLK // SB)
    def _(s):
        base = s * (SB * 128)
        pltpu.sync_copy(src_h.at[pl.ds(base, SB * 128)],
                        tsb.at[pl.ds(0, SB * 128)])
        pltpu.sync_copy(dst_h.at[pl.ds(base, SB * 128)],
                        db.at[pl.ds(0, SB * 128)])
        pltpu.sync_copy(c_h.at[pl.ds(base, SB * 128)],
                        cb.at[pl.ds(0, SB * 128)])

        @pl.loop(0, SB * 8, init_carry=0)
        def cnt(v, n):
            sl = pl.ds(v * 16, 16)
            srcv = tsb[sl]
            dstv = db[sl]
            cv = cb[sl]
            m = (dstv >= lo) & (dstv < lo + HALF)
            gi = srcv * 16 + sidv
            ldv = (dstv - lo) * 16
            plsc.store_compressed(tsb.at[pl.ds(n, 16)], gi, mask=m)
            plsc.store_compressed(db.at[pl.ds(n, 16)], ldv, mask=m)
            plsc.store_compressed(cb.at[pl.ds(n, 16)], cv, mask=m)
            pc = plsc.all_reduce_population_count(m)[0]
            n = n + pc
            cb[pl.ds(n, 16)] = zeros16
            return n

        nch = (cnt + CH - 1) // CH

        @pl.loop(0, CH // 16)
        def _(k):
            pos = cnt + 16 + k * 16

            @pl.when(pos < nch * CH)
            def _():
                cb[pl.ds(pos, 16)] = zeros16

        @pl.when(nch > 0)
        def _():
            fire(0, 0)

        for q in range(SB * 128 // CH):
            @pl.when(q + 1 < nch)
            def _():
                fire(q + 1, (q + 1) % NSLOT)

            @pl.when(q < nch)
            def _():
                drain(q, q % NSLOT)
                finish(q, q % NSLOT)

    pltpu.sync_copy(stripe, out_h.at[cid, sid])


def kernel(x, edge_index, edge_attrs):
    src = edge_index[0]
    dst = edge_index[1]
    pad = E_PAD - E
    srcp = jnp.concatenate([src, jnp.zeros((pad,), _i32)])
    dstp = jnp.concatenate([dst, jnp.full((pad,), N, _i32)])
    attrp = jnp.concatenate([edge_attrs, jnp.full((pad,), -1e30, _f32)])
    ssp, sdp = _stats(srcp, dstp, attrp)
    c = _coeffs(srcp, dstp, attrp, ssp, sdp)
    x_rs = x.reshape(N * 16, FB)
    outp = _accum(x_rs, srcp, dstp, c)
    o = outp.reshape(2, 16, SLAB, FB).transpose(0, 2, 1, 3).reshape(
        2 * SLAB, D)
    return jnp.concatenate([o[:HALF], o[SLAB:SLAB + HALF]], axis=0)


# speedup vs baseline: 1.7062x; 1.0658x over previous
import functools

import jax
import jax.numpy as jnp
from jax import lax
from jax.experimental import pallas as pl
from jax.experimental.pallas import tpu as pltpu
from jax.experimental.pallas import tpu_sc as plsc

N = 10000
E = 160000
D = 256
NP = 10240
E_PAD = 163840
EPW = E_PAD // 32
NBLK = EPW // 128
TBLK = E_PAD // 128
HALF = N // 2
SLAB = 5120
FB = 16
SB = 64
CH = 512
NSLOT = 2

_mesh = plsc.VectorSubcoreMesh(core_axis_name="c", subcore_axis_name="s",
                               num_cores=2, num_subcores=16)
_f32 = jnp.float32
_i32 = jnp.int32


def _rsqrt16(v):
    i = plsc.bitcast(v, _i32)
    i = jnp.full((16,), 0x5F3759DF, _i32) - lax.shift_right_arithmetic(
        i, jnp.full((16,), 1, _i32))
    y = plsc.bitcast(i, _f32)
    for _ in range(3):
        y = y * (1.5 - 0.5 * v * y * y)
    return y


@functools.partial(
    pl.kernel,
    out_type=[
        jax.ShapeDtypeStruct((2, NP), _f32),
        jax.ShapeDtypeStruct((2, NP), _f32),
    ],
    mesh=_mesh,
    compiler_params=pltpu.CompilerParams(needs_layout_passes=False),
    scratch_types=[
        pltpu.VMEM((128,), _i32),
        pltpu.VMEM((128,), _i32),
        pltpu.VMEM((128,), _f32),
        pltpu.VMEM((128,), _f32),
        pltpu.VMEM((NP // 16,), _f32),
        pltpu.VMEM_SHARED((NP,), _f32),
        pltpu.VMEM_SHARED((NP,), _f32),
    ],
)
def _stats(src_h, dst_h, attr_h, ssp_o, sdp_o,
           sb, db, ab, pb, zv, ssrc_sh, sdst_sh):
    cid = lax.axis_index("c")
    sid = lax.axis_index("s")
    w = cid * 16 + sid
    stripe = NP // 16

    zeros16 = jnp.zeros((16,), _f32)

    @pl.loop(0, stripe // 16)
    def _(i):
        zv[pl.ds(i * 16, 16)] = zeros16

    pltpu.sync_copy(zv, ssrc_sh.at[pl.ds(sid * stripe, stripe)])
    pltpu.sync_copy(zv, sdst_sh.at[pl.ds(sid * stripe, stripe)])
    plsc.subcore_barrier()

    @pl.loop(0, NBLK)
    def _(t):
        base = w * EPW + t * 128
        pltpu.sync_copy(src_h.at[pl.ds(base, 128)], sb)
        pltpu.sync_copy(dst_h.at[pl.ds(base, 128)], db)
        pltpu.sync_copy(attr_h.at[pl.ds(base, 128)], ab)

        @pl.loop(0, 8)
        def _(u):
            sl = pl.ds(u * 16, 16)
            pb[sl] = jnp.exp(ab[sl])

        pltpu.sync_copy(pb, ssrc_sh.at[sb], add=True)
        pltpu.sync_copy(pb, sdst_sh.at[db], add=True)

    plsc.subcore_barrier()
    pltpu.sync_copy(ssrc_sh.at[pl.ds(sid * stripe, stripe)],
                    ssp_o.at[cid, pl.ds(sid * stripe, stripe)])
    pltpu.sync_copy(sdst_sh.at[pl.ds(sid * stripe, stripe)],
                    sdp_o.at[cid, pl.ds(sid * stripe, stripe)])


@functools.partial(
    pl.kernel,
    out_type=jax.ShapeDtypeStruct((E_PAD,), _f32),
    mesh=_mesh,
    compiler_params=pltpu.CompilerParams(needs_layout_passes=False),
    scratch_types=[
        pltpu.VMEM((NP,), _f32),
        pltpu.VMEM((NP,), _f32),
        pltpu.VMEM((NP,), _f32),
        pltpu.VMEM((128,), _i32),
        pltpu.VMEM((128,), _i32),
        pltpu.VMEM((128,), _f32),
        pltpu.VMEM((128,), _f32),
    ],
)
def _coeffs(src_h, dst_h, attr_h, ssp_h, sdp_h, c_o,
            ss, sd, tmp, sb, db, ab, cb):
    cid = lax.axis_index("c")
    sid = lax.axis_index("s")
    w = cid * 16 + sid

    pltpu.sync_copy(ssp_h.at[0], ss)
    pltpu.sync_copy(ssp_h.at[1], tmp)

    @pl.loop(0, NP // 16)
    def _(i):
        sl = pl.ds(i * 16, 16)
        ss[sl] = ss[sl] + tmp[sl]

    pltpu.sync_copy(sdp_h.at[0], sd)
    pltpu.sync_copy(sdp_h.at[1], tmp)

    @pl.loop(0, NP // 16)
    def _(i):
        sl = pl.ds(i * 16, 16)
        sd[sl] = sd[sl] + tmp[sl]

    @pl.loop(0, NBLK)
    def _(t):
        base = w * EPW + t * 128
        pltpu.sync_copy(src_h.at[pl.ds(base, 128)], sb)
        pltpu.sync_copy(dst_h.at[pl.ds(base, 128)], db)
        pltpu.sync_copy(attr_h.at[pl.ds(base, 128)], ab)
        for u in range(8):
            sl = pl.ds(u * 16, 16)
            sv = plsc.load_gather(ss, [sb[sl]])
            dv = plsc.load_gather(sd, [db[sl]])
            v = sv * dv
            c = jnp.exp(ab[sl]) * _rsqrt16(v)
            cb[sl] = jnp.where(v > 0.0, c, 0.0)
        pltpu.sync_copy(cb, c_o.at[pl.ds(base, 128)])


@functools.partial(
    pl.kernel,
    out_type=jax.ShapeDtypeStruct((2, 16, SLAB * FB), _f32),
    mesh=_mesh,
    compiler_params=pltpu.CompilerParams(needs_layout_passes=False,
                                         use_tc_tiling_on_sc=False),
    scratch_types=[
        pltpu.VMEM((SLAB * FB,), _f32),
        pltpu.VMEM((NSLOT, CH, FB), _f32),
        pltpu.VMEM((SB * 128 + 544,), _i32),
        pltpu.VMEM((SB * 128 + 544,), _i32),
        pltpu.VMEM((SB * 128 + 544,), _f32),
        pltpu.SemaphoreType.DMA,
        pltpu.SemaphoreType.DMA,
    ],
)
def _accum(x_h, src_h, dst_h, c_h, out_h,
           stripe, g, tsb, db, cb, sem0, sem1):
    cid = lax.axis_index("c")
    sid = lax.axis_index("s")
    lo = cid * HALF
    sems = (sem0, sem1)
    zeros16 = jnp.zeros((16,), _f32)
    iota16 = lax.iota(_i32, 16)
    sidv = jnp.broadcast_to(sid, (16,))

    @pl.loop(0, SLAB)
    def _(i):
        stripe[pl.ds(i * 16, 16)] = zeros16

    def fire(q, p):
        pltpu.async_copy(
            x_h.at[tsb.at[pl.ds(q * CH, CH)]], g.at[p], sems[p])

    def drain(q, p):
        pltpu.make_async_copy(
            x_h.at[tsb.at[pl.ds(q * CH, CH)]], g.at[p], sems[p]).wait()

    def finish(q, p):
        @pl.loop(0, CH // 16)
        def _(u):
            off = q * CH + u * 16
            ldv = db[pl.ds(off, 16)]
            cm = cb[pl.ds(off, 16)]
            for j in range(16):
                idx = jnp.broadcast_to(ldv[j], (16,)) + iota16
                val = g[p, u * 16 + j, pl.ds(0, 16)] * jnp.broadcast_to(
                    cm[j], (16,))
                plsc.addupdate_scatter(stripe, [idx], val)

    @pl.loop(0, TBLK // SB)
    def _(s):
        base = s * (SB * 128)
        pltpu.sync_copy(src_h.at[pl.ds(base, SB * 128)],
                        tsb.at[pl.ds(0, SB * 128)])
        pltpu.sync_copy(dst_h.at[pl.ds(base, SB * 128)],
                        db.at[pl.ds(0, SB * 128)])
        pltpu.sync_copy(c_h.at[pl.ds(base, SB * 128)],
                        cb.at[pl.ds(0, SB * 128)])

        @pl.loop(0, SB * 8, init_carry=0)
        def cnt(v, n):
            sl = pl.ds(v * 16, 16)
            srcv = tsb[sl]
            dstv = db[sl]
            cv = cb[sl]
            m = (dstv >= lo) & (dstv < lo + HALF)
            gi = srcv * 16 + sidv
            ldv = (dstv - lo) * 16
            plsc.store_compressed(tsb.at[pl.ds(n, 16)], gi, mask=m)
            plsc.store_compressed(db.at[pl.ds(n, 16)], ldv, mask=m)
            plsc.store_compressed(cb.at[pl.ds(n, 16)], cv, mask=m)
            pc = plsc.all_reduce_population_count(m)[0]
            return n + pc

        nch = (cnt + CH - 1) // CH

        @pl.loop(0, CH // 16 + 1)
        def _(k):
            pos = cnt + k * 16

            @pl.when(pos < nch * CH)
            def _():
                cb[pl.ds(pos, 16)] = zeros16

        @pl.when(nch > 0)
        def _():
            fire(0, 0)

        for q in range(SB * 128 // CH):
            @pl.when(q + 1 < nch)
            def _():
                fire(q + 1, (q + 1) % NSLOT)

            @pl.when(q < nch)
            def _():
                drain(q, q % NSLOT)
                finish(q, q % NSLOT)

    pltpu.sync_copy(stripe, out_h.at[cid, sid])


def kernel(x, edge_index, edge_attrs):
    src = edge_index[0]
    dst = edge_index[1]
    pad = E_PAD - E
    srcp = jnp.concatenate([src, jnp.zeros((pad,), _i32)])
    dstp = jnp.concatenate([dst, jnp.full((pad,), N, _i32)])
    attrp = jnp.concatenate([edge_attrs, jnp.full((pad,), -1e30, _f32)])
    ssp, sdp = _stats(srcp, dstp, attrp)
    c = _coeffs(srcp, dstp, attrp, ssp, sdp)
    x_rs = x.reshape(N * 16, FB)
    outp = _accum(x_rs, srcp, dstp, c)
    o = outp.reshape(2, 16, SLAB, FB).transpose(0, 2, 1, 3).reshape(
        2 * SLAB, D)
    return jnp.concatenate([o[:HALF], o[SLAB:SLAB + HALF]], axis=0)


# speedup vs baseline: 1.9104x; 1.1196x over previous
import functools

import jax
import jax.numpy as jnp
from jax import lax
from jax.experimental import pallas as pl
from jax.experimental.pallas import tpu as pltpu
from jax.experimental.pallas import tpu_sc as plsc

N = 10000
E = 160000
D = 256
NP = 10240
E_PAD = 163840
EPW = E_PAD // 32
NBLK = EPW // 128
TBLK = E_PAD // 128
HALF = N // 2
SLAB = 5120
FB = 16
SB = 64
CH = 512
NSLOT = 2

_mesh = plsc.VectorSubcoreMesh(core_axis_name="c", subcore_axis_name="s",
                               num_cores=2, num_subcores=16)
_f32 = jnp.float32
_i32 = jnp.int32


def _rsqrt16(v):
    i = plsc.bitcast(v, _i32)
    i = jnp.full((16,), 0x5F3759DF, _i32) - lax.shift_right_arithmetic(
        i, jnp.full((16,), 1, _i32))
    y = plsc.bitcast(i, _f32)
    for _ in range(3):
        y = y * (1.5 - 0.5 * v * y * y)
    return y


@functools.partial(
    pl.kernel,
    out_type=[
        jax.ShapeDtypeStruct((2, NP), _f32),
        jax.ShapeDtypeStruct((2, NP), _f32),
    ],
    mesh=_mesh,
    compiler_params=pltpu.CompilerParams(needs_layout_passes=False),
    scratch_types=[
        pltpu.VMEM((EPW,), _i32),
        pltpu.VMEM((EPW,), _i32),
        pltpu.VMEM((EPW,), _f32),
        pltpu.VMEM((128,), _i32),
        pltpu.VMEM((NP // 16,), _f32),
        pltpu.VMEM_SHARED((NP,), _f32),
        pltpu.VMEM_SHARED((NP,), _f32),
    ],
)
def _stats(src_h, dst_h, attr_h, ssp_o, sdp_o,
           sb, db, ab, ib, zv, ssrc_sh, sdst_sh):
    cid = lax.axis_index("c")
    sid = lax.axis_index("s")
    w = cid * 16 + sid
    stripe = NP // 16

    zeros16 = jnp.zeros((16,), _f32)

    @pl.loop(0, stripe // 16)
    def _(i):
        zv[pl.ds(i * 16, 16)] = zeros16

    pltpu.sync_copy(zv, ssrc_sh.at[pl.ds(sid * stripe, stripe)])
    pltpu.sync_copy(zv, sdst_sh.at[pl.ds(sid * stripe, stripe)])
    plsc.subcore_barrier()

    pltpu.sync_copy(src_h.at[pl.ds(w * EPW, EPW)], sb)
    pltpu.sync_copy(dst_h.at[pl.ds(w * EPW, EPW)], db)
    pltpu.sync_copy(attr_h.at[pl.ds(w * EPW, EPW)], ab)

    @pl.loop(0, EPW // 16)
    def _(u):
        sl = pl.ds(u * 16, 16)
        ab[sl] = jnp.exp(ab[sl])

    @pl.loop(0, NBLK)
    def _(t):
        base = t * 128

        @pl.loop(0, 8)
        def _(u):
            ib[pl.ds(u * 16, 16)] = sb[pl.ds(base + u * 16, 16)]

        pltpu.sync_copy(ab.at[pl.ds(base, 128)], ssrc_sh.at[ib], add=True)

        @pl.loop(0, 8)
        def _(u):
            ib[pl.ds(u * 16, 16)] = db[pl.ds(base + u * 16, 16)]

        pltpu.sync_copy(ab.at[pl.ds(base, 128)], sdst_sh.at[ib], add=True)

    plsc.subcore_barrier()
    pltpu.sync_copy(ssrc_sh.at[pl.ds(sid * stripe, stripe)],
                    ssp_o.at[cid, pl.ds(sid * stripe, stripe)])
    pltpu.sync_copy(sdst_sh.at[pl.ds(sid * stripe, stripe)],
                    sdp_o.at[cid, pl.ds(sid * stripe, stripe)])


@functools.partial(
    pl.kernel,
    out_type=jax.ShapeDtypeStruct((E_PAD,), _f32),
    mesh=_mesh,
    compiler_params=pltpu.CompilerParams(needs_layout_passes=False),
    scratch_types=[
        pltpu.VMEM((NP,), _f32),
        pltpu.VMEM((NP,), _f32),
        pltpu.VMEM((NP,), _f32),
        pltpu.VMEM((EPW,), _i32),
        pltpu.VMEM((EPW,), _i32),
        pltpu.VMEM((EPW,), _f32),
        pltpu.VMEM((EPW,), _f32),
    ],
)
def _coeffs(src_h, dst_h, attr_h, ssp_h, sdp_h, c_o,
            ss, sd, tmp, sb, db, ab, cb):
    cid = lax.axis_index("c")
    sid = lax.axis_index("s")
    w = cid * 16 + sid

    pltpu.sync_copy(ssp_h.at[0], ss)
    pltpu.sync_copy(ssp_h.at[1], tmp)

    @pl.loop(0, NP // 16)
    def _(i):
        sl = pl.ds(i * 16, 16)
        ss[sl] = ss[sl] + tmp[sl]

    pltpu.sync_copy(sdp_h.at[0], sd)
    pltpu.sync_copy(sdp_h.at[1], tmp)

    @pl.loop(0, NP // 16)
    def _(i):
        sl = pl.ds(i * 16, 16)
        sd[sl] = sd[sl] + tmp[sl]

    pltpu.sync_copy(src_h.at[pl.ds(w * EPW, EPW)], sb)
    pltpu.sync_copy(dst_h.at[pl.ds(w * EPW, EPW)], db)
    pltpu.sync_copy(attr_h.at[pl.ds(w * EPW, EPW)], ab)

    @pl.loop(0, EPW // 16)
    def _(u):
        sl = pl.ds(u * 16, 16)
        sv = plsc.load_gather(ss, [sb[sl]])
        dv = plsc.load_gather(sd, [db[sl]])
        v = sv * dv
        c = jnp.exp(ab[sl]) * _rsqrt16(v)
        cb[sl] = jnp.where(v > 0.0, c, 0.0)

    pltpu.sync_copy(cb, c_o.at[pl.ds(w * EPW, EPW)])


@functools.partial(
    pl.kernel,
    out_type=jax.ShapeDtypeStruct((2, 16, SLAB * FB), _f32),
    mesh=_mesh,
    compiler_params=pltpu.CompilerParams(needs_layout_passes=False,
                                         use_tc_tiling_on_sc=False),
    scratch_types=[
        pltpu.VMEM((SLAB * FB,), _f32),
        pltpu.VMEM((NSLOT, CH, FB), _f32),
        pltpu.VMEM((SB * 128 + 544,), _i32),
        pltpu.VMEM((SB * 128 + 544,), _i32),
        pltpu.VMEM((SB * 128 + 544,), _f32),
        pltpu.SemaphoreType.DMA,
        pltpu.SemaphoreType.DMA,
    ],
)
def _accum(x_h, src_h, dst_h, c_h, out_h,
           stripe, g, tsb, db, cb, sem0, sem1):
    cid = lax.axis_index("c")
    sid = lax.axis_index("s")
    lo = cid * HALF
    sems = (sem0, sem1)
    zeros16 = jnp.zeros((16,), _f32)
    iota16 = lax.iota(_i32, 16)
    sidv = jnp.broadcast_to(sid, (16,))

    @pl.loop(0, SLAB)
    def _(i):
        stripe[pl.ds(i * 16, 16)] = zeros16

    def fire(q, p):
        pltpu.async_copy(
            x_h.at[tsb.at[pl.ds(q * CH, CH)]], g.at[p], sems[p])

    def drain(q, p):
        pltpu.make_async_copy(
            x_h.at[tsb.at[pl.ds(q * CH, CH)]], g.at[p], sems[p]).wait()

    def finish(q, p):
        @pl.loop(0, CH // 16)
        def _(u):
            off = q * CH + u * 16
            ldv = db[pl.ds(off, 16)]
            cm = cb[pl.ds(off, 16)]
            for j in range(16):
                idx = jnp.broadcast_to(ldv[j], (16,)) + iota16
                val = g[p, u * 16 + j, pl.ds(0, 16)] * jnp.broadcast_to(
                    cm[j], (16,))
                plsc.addupdate_scatter(stripe, [idx], val)

    @pl.loop(0, TBLK // SB)
    def _(s):
        base = s * (SB * 128)
        pltpu.sync_copy(src_h.at[pl.ds(base, SB * 128)],
                        tsb.at[pl.ds(0, SB * 128)])
        pltpu.sync_copy(dst_h.at[pl.ds(base, SB * 128)],
                        db.at[pl.ds(0, SB * 128)])
        pltpu.sync_copy(c_h.at[pl.ds(base, SB * 128)],
                        cb.at[pl.ds(0, SB * 128)])

        @pl.loop(0, SB * 8, init_carry=0)
        def cnt(v, n):
            sl = pl.ds(v * 16, 16)
            srcv = tsb[sl]
            dstv = db[sl]
            cv = cb[sl]
            m = (dstv >= lo) & (dstv < lo + HALF)
            gi = srcv * 16 + sidv
            ldv = (dstv - lo) * 16
            plsc.store_compressed(tsb.at[pl.ds(n, 16)], gi, mask=m)
            plsc.store_compressed(db.at[pl.ds(n, 16)], ldv, mask=m)
            plsc.store_compressed(cb.at[pl.ds(n, 16)], cv, mask=m)
            pc = plsc.all_reduce_population_count(m)[0]
            return n + pc

        nch = (cnt + CH - 1) // CH

        @pl.loop(0, CH // 16 + 1)
        def _(k):
            pos = cnt + k * 16

            @pl.when(pos < nch * CH)
            def _():
                cb[pl.ds(pos, 16)] = zeros16

        @pl.when(nch > 0)
        def _():
            fire(0, 0)

        for q in range(SB * 128 // CH):
            @pl.when(q + 1 < nch)
            def _():
                fire(q + 1, (q + 1) % NSLOT)

            @pl.when(q < nch)
            def _():
                drain(q, q % NSLOT)
                finish(q, q % NSLOT)

    pltpu.sync_copy(stripe, out_h.at[cid, sid])


def kernel(x, edge_index, edge_attrs):
    src = edge_index[0]
    dst = edge_index[1]
    pad = E_PAD - E
    srcp = jnp.concatenate([src, jnp.zeros((pad,), _i32)])
    dstp = jnp.concatenate([dst, jnp.full((pad,), N, _i32)])
    attrp = jnp.concatenate([edge_attrs, jnp.full((pad,), -1e30, _f32)])
    ssp, sdp = _stats(srcp, dstp, attrp)
    c = _coeffs(srcp, dstp, attrp, ssp, sdp)
    x_rs = x.reshape(N * 16, FB)
    outp = _accum(x_rs, srcp, dstp, c)
    o = outp.reshape(2, 16, SLAB, FB).transpose(0, 2, 1, 3).reshape(
        2 * SLAB, D)
    return jnp.concatenate([o[:HALF], o[SLAB:SLAB + HALF]], axis=0)


# speedup vs baseline: 1.9315x; 1.0111x over previous
import functools

import jax
import jax.numpy as jnp
from jax import lax
from jax.experimental import pallas as pl
from jax.experimental.pallas import tpu as pltpu
from jax.experimental.pallas import tpu_sc as plsc

N = 10000
E = 160000
D = 256
NP = 10240
E_PAD = 163840
EPW = E_PAD // 32
NBLK = EPW // 128
TBLK = E_PAD // 128
HALF = N // 2
SLAB = 5120
FB = 16
SB = 64
CH = 384
NSLOT = 3

_mesh = plsc.VectorSubcoreMesh(core_axis_name="c", subcore_axis_name="s",
                               num_cores=2, num_subcores=16)
_f32 = jnp.float32
_i32 = jnp.int32


def _rsqrt16(v):
    i = plsc.bitcast(v, _i32)
    i = jnp.full((16,), 0x5F3759DF, _i32) - lax.shift_right_arithmetic(
        i, jnp.full((16,), 1, _i32))
    y = plsc.bitcast(i, _f32)
    for _ in range(3):
        y = y * (1.5 - 0.5 * v * y * y)
    return y


@functools.partial(
    pl.kernel,
    out_type=[
        jax.ShapeDtypeStruct((2, NP), _f32),
        jax.ShapeDtypeStruct((2, NP), _f32),
    ],
    mesh=_mesh,
    compiler_params=pltpu.CompilerParams(needs_layout_passes=False),
    scratch_types=[
        pltpu.VMEM((EPW,), _i32),
        pltpu.VMEM((EPW,), _i32),
        pltpu.VMEM((EPW,), _f32),
        pltpu.VMEM((128,), _i32),
        pltpu.VMEM((NP // 16,), _f32),
        pltpu.VMEM_SHARED((NP,), _f32),
        pltpu.VMEM_SHARED((NP,), _f32),
    ],
)
def _stats(src_h, dst_h, attr_h, ssp_o, sdp_o,
           sb, db, ab, ib, zv, ssrc_sh, sdst_sh):
    cid = lax.axis_index("c")
    sid = lax.axis_index("s")
    w = cid * 16 + sid
    stripe = NP // 16

    zeros16 = jnp.zeros((16,), _f32)

    @pl.loop(0, stripe // 16)
    def _(i):
        zv[pl.ds(i * 16, 16)] = zeros16

    pltpu.sync_copy(zv, ssrc_sh.at[pl.ds(sid * stripe, stripe)])
    pltpu.sync_copy(zv, sdst_sh.at[pl.ds(sid * stripe, stripe)])
    plsc.subcore_barrier()

    pltpu.sync_copy(src_h.at[pl.ds(w * EPW, EPW)], sb)
    pltpu.sync_copy(dst_h.at[pl.ds(w * EPW, EPW)], db)
    pltpu.sync_copy(attr_h.at[pl.ds(w * EPW, EPW)], ab)

    @pl.loop(0, EPW // 16)
    def _(u):
        sl = pl.ds(u * 16, 16)
        ab[sl] = jnp.exp(ab[sl])

    @pl.loop(0, NBLK)
    def _(t):
        base = t * 128

        @pl.loop(0, 8)
        def _(u):
            ib[pl.ds(u * 16, 16)] = sb[pl.ds(base + u * 16, 16)]

        pltpu.sync_copy(ab.at[pl.ds(base, 128)], ssrc_sh.at[ib], add=True)

        @pl.loop(0, 8)
        def _(u):
            ib[pl.ds(u * 16, 16)] = db[pl.ds(base + u * 16, 16)]

        pltpu.sync_copy(ab.at[pl.ds(base, 128)], sdst_sh.at[ib], add=True)

    plsc.subcore_barrier()
    pltpu.sync_copy(ssrc_sh.at[pl.ds(sid * stripe, stripe)],
                    ssp_o.at[cid, pl.ds(sid * stripe, stripe)])
    pltpu.sync_copy(sdst_sh.at[pl.ds(sid * stripe, stripe)],
                    sdp_o.at[cid, pl.ds(sid * stripe, stripe)])


@functools.partial(
    pl.kernel,
    out_type=jax.ShapeDtypeStruct((E_PAD,), _f32),
    mesh=_mesh,
    compiler_params=pltpu.CompilerParams(needs_layout_passes=False),
    scratch_types=[
        pltpu.VMEM((NP,), _f32),
        pltpu.VMEM((NP,), _f32),
        pltpu.VMEM((NP,), _f32),
        pltpu.VMEM((EPW,), _i32),
        pltpu.VMEM((EPW,), _i32),
        pltpu.VMEM((EPW,), _f32),
        pltpu.VMEM((EPW,), _f32),
    ],
)
def _coeffs(src_h, dst_h, attr_h, ssp_h, sdp_h, c_o,
            ss, sd, tmp, sb, db, ab, cb):
    cid = lax.axis_index("c")
    sid = lax.axis_index("s")
    w = cid * 16 + sid

    pltpu.sync_copy(ssp_h.at[0], ss)
    pltpu.sync_copy(ssp_h.at[1], tmp)

    @pl.loop(0, NP // 16)
    def _(i):
        sl = pl.ds(i * 16, 16)
        ss[sl] = ss[sl] + tmp[sl]

    pltpu.sync_copy(sdp_h.at[0], sd)
    pltpu.sync_copy(sdp_h.at[1], tmp)

    @pl.loop(0, NP // 16)
    def _(i):
        sl = pl.ds(i * 16, 16)
        sd[sl] = sd[sl] + tmp[sl]

    pltpu.sync_copy(src_h.at[pl.ds(w * EPW, EPW)], sb)
    pltpu.sync_copy(dst_h.at[pl.ds(w * EPW, EPW)], db)
    pltpu.sync_copy(attr_h.at[pl.ds(w * EPW, EPW)], ab)

    @pl.loop(0, EPW // 16)
    def _(u):
        sl = pl.ds(u * 16, 16)
        sv = plsc.load_gather(ss, [sb[sl]])
        dv = plsc.load_gather(sd, [db[sl]])
        v = sv * dv
        c = jnp.exp(ab[sl]) * _rsqrt16(v)
        cb[sl] = jnp.where(v > 0.0, c, 0.0)

    pltpu.sync_copy(cb, c_o.at[pl.ds(w * EPW, EPW)])


@functools.partial(
    pl.kernel,
    out_type=jax.ShapeDtypeStruct((2, 16, SLAB * FB), _f32),
    mesh=_mesh,
    compiler_params=pltpu.CompilerParams(needs_layout_passes=False,
                                         use_tc_tiling_on_sc=False),
    scratch_types=[
        pltpu.VMEM((SLAB * FB,), _f32),
        pltpu.VMEM((NSLOT, CH, FB), _f32),
        pltpu.VMEM((SB * 128 + 1024,), _i32),
        pltpu.VMEM((SB * 128 + 1024,), _i32),
        pltpu.VMEM((SB * 128 + 1024,), _f32),
        pltpu.SemaphoreType.DMA,
        pltpu.SemaphoreType.DMA,
        pltpu.SemaphoreType.DMA,
    ],
)
def _accum(x_h, src_h, dst_h, c_h, out_h,
           stripe, g, tsb, db, cb, sem0, sem1, sem2):
    cid = lax.axis_index("c")
    sid = lax.axis_index("s")
    lo = cid * HALF
    sems = (sem0, sem1, sem2)
    zeros16 = jnp.zeros((16,), _f32)
    iota16 = lax.iota(_i32, 16)
    sidv = jnp.broadcast_to(sid, (16,))

    @pl.loop(0, SLAB)
    def _(i):
        stripe[pl.ds(i * 16, 16)] = zeros16

    def fire(q, p):
        pltpu.async_copy(
            x_h.at[tsb.at[pl.ds(q * CH, CH)]], g.at[p], sems[p])

    def drain(q, p):
        pltpu.make_async_copy(
            x_h.at[tsb.at[pl.ds(q * CH, CH)]], g.at[p], sems[p]).wait()

    def finish(q, p):
        @pl.loop(0, CH // 16)
        def _(u):
            off = q * CH + u * 16
            ldv = db[pl.ds(off, 16)]
            cm = cb[pl.ds(off, 16)]
            for j in range(16):
                idx = jnp.broadcast_to(ldv[j], (16,)) + iota16
                val = g[p, u * 16 + j, pl.ds(0, 16)] * jnp.broadcast_to(
                    cm[j], (16,))
                plsc.addupdate_scatter(stripe, [idx], val)

    @pl.loop(0, TBLK // SB)
    def _(s):
        base = s * (SB * 128)
        pltpu.sync_copy(src_h.at[pl.ds(base, SB * 128)],
                        tsb.at[pl.ds(0, SB * 128)])
        pltpu.sync_copy(dst_h.at[pl.ds(base, SB * 128)],
                        db.at[pl.ds(0, SB * 128)])
        pltpu.sync_copy(c_h.at[pl.ds(base, SB * 128)],
                        cb.at[pl.ds(0, SB * 128)])

        @pl.loop(0, SB * 8, init_carry=0)
        def cnt(v, n):
            sl = pl.ds(v * 16, 16)
            srcv = tsb[sl]
            dstv = db[sl]
            cv = cb[sl]
            m = (dstv >= lo) & (dstv < lo + HALF)
            gi = srcv * 16 + sidv
            ldv = (dstv - lo) * 16
            plsc.store_compressed(tsb.at[pl.ds(n, 16)], gi, mask=m)
            plsc.store_compressed(db.at[pl.ds(n, 16)], ldv, mask=m)
            plsc.store_compressed(cb.at[pl.ds(n, 16)], cv, mask=m)
            pc = plsc.all_reduce_population_count(m)[0]
            return n + pc

        nch = (cnt + CH - 1) // CH

        @pl.loop(0, CH // 16 + 1)
        def _(k):
            pos = cnt + k * 16

            @pl.when(pos < nch * CH)
            def _():
                cb[pl.ds(pos, 16)] = zeros16

        for q0 in range(NSLOT - 1):
            @pl.when(q0 < nch)
            def _():
                fire(q0, q0)

        for q in range((SB * 128 + CH - 1) // CH):
            @pl.when(q + NSLOT - 1 < nch)
            def _():
                fire(q + NSLOT - 1, (q + NSLOT - 1) % NSLOT)

            @pl.when(q < nch)
            def _():
                drain(q, q % NSLOT)
                finish(q, q % NSLOT)

    pltpu.sync_copy(stripe, out_h.at[cid, sid])


def kernel(x, edge_index, edge_attrs):
    src = edge_index[0]
    dst = edge_index[1]
    pad = E_PAD - E
    srcp = jnp.concatenate([src, jnp.zeros((pad,), _i32)])
    dstp = jnp.concatenate([dst, jnp.full((pad,), N, _i32)])
    attrp = jnp.concatenate([edge_attrs, jnp.full((pad,), -1e30, _f32)])
    ssp, sdp = _stats(srcp, dstp, attrp)
    c = _coeffs(srcp, dstp, attrp, ssp, sdp)
    x_rs = x.reshape(N * 16, FB)
    outp = _accum(x_rs, srcp, dstp, c)
    o = outp.reshape(2, 16, SLAB, FB).transpose(0, 2, 1, 3).reshape(
        2 * SLAB, D)
    return jnp.concatenate([o[:HALF], o[SLAB:SLAB + HALF]], axis=0)
